# baseline (speedup 1.0000x reference)
import functools

import jax
import jax.numpy as jnp
from jax import lax
from jax.experimental import pallas as pl
from jax.experimental.pallas import tpu as pltpu
from jax.experimental.pallas import tpu_sc as plsc

N_F = 64
BATCH = 16384

NC = 2
NS = 16
L = 16
NW = NC * NS
B_PER_W = BATCH // NW
CHUNKS = B_PER_W // L


def _sc_body(user_hbm, item_hbm, gm_hbm, wput_hbm, wpit_hbm, bu_hbm, bi_hbm,
             out_hbm,
             uidx_v, iidx_v, ufeat_v, ifeat_v, bu_v, bi_v, gm_v, out_v, sem):
    wid = lax.axis_index("s") * NC + lax.axis_index("c")
    base = wid * B_PER_W

    pltpu.sync_copy(user_hbm.at[pl.ds(base, B_PER_W)], uidx_v)
    pltpu.sync_copy(item_hbm.at[pl.ds(base, B_PER_W)], iidx_v)
    pltpu.sync_copy(gm_hbm, gm_v)

    cb = pltpu.async_copy(bu_hbm.at[uidx_v], bu_v, sem)
    ci = pltpu.async_copy(bi_hbm.at[iidx_v], bi_v, sem)

    cb.wait()
    ci.wait()

    LAG = 8
    descs = []
    for j in range(N_F):
        descs.append(pltpu.async_copy(wput_hbm.at[j].at[uidx_v],
                                      ufeat_v.at[j], sem))
        descs.append(pltpu.async_copy(wpit_hbm.at[j].at[iidx_v],
                                      ifeat_v.at[j], sem))
        while len(descs) > LAG:
            descs.pop(0).wait()
    for d in descs:
        d.wait()

    gm = gm_v[...]

    def chunk(c, carry):
        col = c * L
        acc = jnp.zeros((L,), jnp.float32)
        for j in range(N_F):
            u = ufeat_v[j, pl.ds(col, L)]
            it = ifeat_v[j, pl.ds(col, L)]
            acc = acc + u * it
        out_v[pl.ds(col, L)] = (acc + bu_v[pl.ds(col, L)]
                                + bi_v[pl.ds(col, L)] + gm)
        return carry

    lax.fori_loop(0, CHUNKS, chunk, 0)
    pltpu.sync_copy(out_v, out_hbm.at[pl.ds(base, B_PER_W)])


@jax.jit
def _sc_call(user, item, gm16, WPUT, WPIT, BU, BI):
    mesh = plsc.VectorSubcoreMesh(core_axis_name="c", subcore_axis_name="s")
    f = pl.kernel(
        _sc_body,
        out_type=jax.ShapeDtypeStruct((BATCH,), jnp.float32),
        mesh=mesh,
        scratch_types=[
            pltpu.VMEM((B_PER_W,), jnp.int32),
            pltpu.VMEM((B_PER_W,), jnp.int32),
            pltpu.VMEM((N_F, B_PER_W), jnp.float32),
            pltpu.VMEM((N_F, B_PER_W), jnp.float32),
            pltpu.VMEM((B_PER_W,), jnp.float32),
            pltpu.VMEM((B_PER_W,), jnp.float32),
            pltpu.VMEM((L,), jnp.float32),
            pltpu.VMEM((B_PER_W,), jnp.float32),
            pltpu.SemaphoreType.DMA,
        ],
        compiler_params=pltpu.CompilerParams(
            needs_layout_passes=False, use_tc_tiling_on_sc=False),
    )
    return f(user, item, gm16, WPUT, WPIT, BU, BI)


def kernel(user, item, global_mean, WPU, WPI, BU, BI):
    user = user.astype(jnp.int32)
    item = item.astype(jnp.int32)
    gm16 = jnp.broadcast_to(global_mean.astype(jnp.float32), (L,))
    return _sc_call(user, item, gm16, WPU.T, WPI.T, BU, BI)


# speedup vs baseline: 2.6948x; 2.6948x over previous
import functools

import jax
import jax.numpy as jnp
from jax import lax
from jax.experimental import pallas as pl
from jax.experimental.pallas import tpu as pltpu
from jax.experimental.pallas import tpu_sc as plsc

N_ROWS = 1000000
N_F = 64
BATCH = 16384

NC = 2
NS = 16
L = 16
NW = NC * NS
B_PER_W = BATCH // NW

MAIN_ROWS = 999936
NBLK = MAIN_ROWS // 128
BLK_PER_W = (NBLK + NW - 1) // NW
PACKED_ROWS = N_ROWS // 2
TAIL_PACKED = (N_ROWS - MAIN_ROWS) // 2
HALF = B_PER_W // 2


def _convert_body(wput_hbm, wpit_hbm, tailu_hbm, taili_hbm,
                  outu_hbm, outi_hbm,
                  blk_v, oblk_v, tail_v, lanes_ref):
    wid = lax.axis_index("s") * NC + lax.axis_index("c")
    lanes = lax.iota(jnp.int32, L)

    def do_block(src_hbm, dst_hbm, J):
        c0 = pl.multiple_of(J * 128, 128)
        pltpu.sync_copy(src_hbm.at[:, pl.ds(c0, 128)], blk_v)
        def row(k, carry):
            for s in range(2):
                col = 2 * k + s
                cols = jnp.full((L,), 0, jnp.int32) + col
                for g in range(N_F // L):
                    feats = g * L + lanes
                    v = plsc.load_gather(blk_v, [feats, cols])
                    oblk_v[k, pl.ds(s * N_F + g * L, L)] = v
            return carry
        lax.fori_loop(0, N_F, row, 0)
        r0 = pl.multiple_of(J * 64, 8)
        pltpu.sync_copy(oblk_v, dst_hbm.at[pl.ds(r0, 64), :])

    def step(t, carry):
        J = t * NW + wid
        @pl.when(J < NBLK)
        def _():
            do_block(wput_hbm, outu_hbm, J)
            do_block(wpit_hbm, outi_hbm, J)
        return carry

    lax.fori_loop(0, BLK_PER_W, step, 0)

    @pl.when(wid == 0)
    def _():
        pltpu.sync_copy(tailu_hbm, tail_v)
        pltpu.sync_copy(tail_v, outu_hbm.at[pl.ds(PACKED_ROWS - TAIL_PACKED,
                                                  TAIL_PACKED), :])
    @pl.when(wid == 1)
    def _():
        pltpu.sync_copy(taili_hbm, tail_v)
        pltpu.sync_copy(tail_v, outi_hbm.at[pl.ds(PACKED_ROWS - TAIL_PACKED,
                                                  TAIL_PACKED), :])


def _dots_body(user_hbm, item_hbm, gm_hbm, pku_hbm, pki_hbm, bu_hbm, bi_hbm,
               out_hbm,
               uidx_v, iidx_v, upk_v, ipk_v, urows_v, irows_v,
               bu_v, bi_v, gm_v, out_v, sem):
    wid = lax.axis_index("s") * NC + lax.axis_index("c")
    base = wid * B_PER_W
    lanes = lax.iota(jnp.int32, L)

    pltpu.sync_copy(user_hbm.at[pl.ds(base, B_PER_W)], uidx_v)
    pltpu.sync_copy(item_hbm.at[pl.ds(base, B_PER_W)], iidx_v)
    pltpu.sync_copy(gm_hbm, gm_v)

    cb = pltpu.async_copy(bu_hbm.at[uidx_v], bu_v, sem)
    ci = pltpu.async_copy(bi_hbm.at[iidx_v], bi_v, sem)

    def mkpk(c, carry):
        u = uidx_v[pl.ds(c * L, L)]
        upk_v[pl.ds(c * L, L)] = lax.shift_right_logical(u, 1)
        it = iidx_v[pl.ds(c * L, L)]
        ipk_v[pl.ds(c * L, L)] = lax.shift_right_logical(it, 1)
        return carry
    lax.fori_loop(0, B_PER_W // L, mkpk, 0)

    cb.wait()
    ci.wait()
    gm = gm_v[...]

    for h in range(2):
        hb = h * HALF
        cu = pltpu.async_copy(pku_hbm.at[upk_v.at[pl.ds(hb, HALF)]],
                              urows_v, sem)
        ci2 = pltpu.async_copy(pki_hbm.at[ipk_v.at[pl.ds(hb, HALF)]],
                               irows_v, sem)
        cu.wait()
        ci2.wait()

        def group(g, carry):
            k0 = g * L
            rows = k0 + lanes
            uo = (uidx_v[pl.ds(hb + k0, L)] & 1) * N_F
            io = (iidx_v[pl.ds(hb + k0, L)] & 1) * N_F
            acc = jnp.zeros((L,), jnp.float32)
            for j in range(N_F):
                uv = plsc.load_gather(urows_v, [rows, uo + j])
                iv = plsc.load_gather(irows_v, [rows, io + j])
                acc = acc + uv * iv
            out_v[pl.ds(hb + k0, L)] = (acc + bu_v[pl.ds(hb + k0, L)]
                                        + bi_v[pl.ds(hb + k0, L)] + gm)
            return carry
        lax.fori_loop(0, HALF // L, group, 0)

    pltpu.sync_copy(out_v, out_hbm.at[pl.ds(base, B_PER_W)])


@jax.jit
def _run(user, item, gm16, WPUT, WPIT, tail_u, tail_i, BU, BI):
    mesh = plsc.VectorSubcoreMesh(core_axis_name="c", subcore_axis_name="s")
    convert = pl.kernel(
        _convert_body,
        out_type=(jax.ShapeDtypeStruct((PACKED_ROWS, 128), jnp.float32),
                  jax.ShapeDtypeStruct((PACKED_ROWS, 128), jnp.float32)),
        mesh=mesh,
        scratch_types=[
            pltpu.VMEM((N_F, 128), jnp.float32),
            pltpu.VMEM((N_F, 128), jnp.float32),
            pltpu.VMEM((TAIL_PACKED, 128), jnp.float32),
            pltpu.VMEM((L,), jnp.int32),
        ],
        compiler_params=pltpu.CompilerParams(needs_layout_passes=False),
    )
    pku, pki = convert(WPUT, WPIT, tail_u, tail_i)

    dots = pl.kernel(
        _dots_body,
        out_type=jax.ShapeDtypeStruct((BATCH,), jnp.float32),
        mesh=mesh,
        scratch_types=[
            pltpu.VMEM((B_PER_W,), jnp.int32),
            pltpu.VMEM((B_PER_W,), jnp.int32),
            pltpu.VMEM((B_PER_W,), jnp.int32),
            pltpu.VMEM((B_PER_W,), jnp.int32),
            pltpu.VMEM((HALF, 128), jnp.float32),
            pltpu.VMEM((HALF, 128), jnp.float32),
            pltpu.VMEM((B_PER_W,), jnp.float32),
            pltpu.VMEM((B_PER_W,), jnp.float32),
            pltpu.VMEM((L,), jnp.float32),
            pltpu.VMEM((B_PER_W,), jnp.float32),
            pltpu.SemaphoreType.DMA,
        ],
        compiler_params=pltpu.CompilerParams(needs_layout_passes=False),
    )
    return dots(user, item, gm16, pku, pki, BU, BI)


def kernel(user, item, global_mean, WPU, WPI, BU, BI):
    user = user.astype(jnp.int32)
    item = item.astype(jnp.int32)
    gm16 = jnp.broadcast_to(global_mean.astype(jnp.float32), (L,))
    WPUT = WPU.T
    WPIT = WPI.T
    tail_u = WPU[MAIN_ROWS:].reshape(TAIL_PACKED, 128)
    tail_i = WPI[MAIN_ROWS:].reshape(TAIL_PACKED, 128)
    return _run(user, item, gm16, WPUT, WPIT, tail_u, tail_i, BU, BI)
